# baseline (speedup 1.0000x reference)
import functools

import jax
import jax.numpy as jnp
from jax import lax
from jax.experimental import pallas as pl
from jax.experimental.pallas import tpu as pltpu
from jax.experimental.pallas import tpu_sc as plsc

B, C, H, W = 32, 7, 64, 64
GH, GW = 512, 512
BLK = 8
HW = H * W
PR = HW // 128
NC, NS, L = 2, 16, 16
SCK = 16
TPI = (NC * NS) // SCK
ROWS_T = GH // TPI
CHUNK = 64
NCHUNK = ROWS_T // CHUNK
BR_PER_CHUNK = CHUNK // BLK
VPR = GW // L
THRESH = (1, 2, 4, 8, 16, 32, 49)


def _sc_cls_body(gt_hbm, cls_hbm, in_buf0, in_buf1, rowsum, out_buf,
                 sem0, sem1):
    wid = lax.axis_index("s") * NC + lax.axis_index("c")
    img = wid // TPI
    half = wid % TPI
    lanes = lax.iota(jnp.int32, L)
    bufs = (in_buf0, in_buf1)
    sems = (sem0, sem1)

    def src_rows(chunk):
        return gt_hbm.at[img, 0, pl.ds(half * ROWS_T + chunk * CHUNK, CHUNK)]

    def compute_chunk(chunk, buf):
        def br_body(br, c2):
            for v0 in range(0, VPR, 8):
                sums = []
                for v in range(v0, v0 + 8):
                    xs = [buf[br * BLK + r, pl.ds(v * L, L)]
                          for r in range(BLK)]
                    while len(xs) > 1:
                        xs = [xs[i] + xs[i + 1] for i in range(0, len(xs), 2)]
                    sums.append(xs[0])
                for dv, s in enumerate(sums):
                    rowsum[pl.ds((v0 + dv) * L, L)] = s
            brl = chunk * BR_PER_CHUNK + br
            orow = brl // 2
            ocol0 = (brl % 2) * W
            for g in range(W // L):
                idx0 = lanes * BLK + g * (L * BLK)
                gs = [plsc.load_gather(rowsum, [idx0 + j]) for j in range(BLK)]
                while len(gs) > 1:
                    gs = [gs[i] + gs[i + 1] for i in range(0, len(gs), 2)]
                acc = gs[0]
                cls = (acc >= THRESH[0]).astype(jnp.int32)
                for t in THRESH[1:]:
                    cls = cls + (acc >= t).astype(jnp.int32)
                out_buf[orow, pl.ds(ocol0 + g * L, L)] = cls
            return c2

        lax.fori_loop(0, BR_PER_CHUNK, br_body, 0)

    pltpu.async_copy(src_rows(0), bufs[0], sems[0])
    pltpu.async_copy(src_rows(1), bufs[1], sems[1])
    for chunk in range(NCHUNK):
        b = chunk % 2
        pltpu.make_async_copy(src_rows(0), bufs[b], sems[b]).wait()
        compute_chunk(chunk, bufs[b])
        if chunk + 2 < NCHUNK:
            pltpu.async_copy(src_rows(chunk + 2), bufs[b], sems[b])
    pltpu.sync_copy(out_buf,
                    cls_hbm.at[img, pl.ds(half * (PR // TPI), PR // TPI)])


@jax.jit
def _sc_cls(gt4):
    mesh = plsc.VectorSubcoreMesh(core_axis_name="c", subcore_axis_name="s",
                                  num_cores=NC, num_subcores=NS)
    return pl.kernel(
        _sc_cls_body,
        out_type=jax.ShapeDtypeStruct((B, PR, 128), jnp.int32),
        mesh=mesh,
        compiler_params=pltpu.CompilerParams(needs_layout_passes=False),
        scratch_types=[
            pltpu.VMEM((CHUNK, GW), jnp.int32),
            pltpu.VMEM((CHUNK, GW), jnp.int32),
            pltpu.VMEM((GW,), jnp.int32),
            pltpu.VMEM((PR // TPI, 128), jnp.int32),
            pltpu.SemaphoreType.DMA,
            pltpu.SemaphoreType.DMA,
        ],
    )(gt4)


def _tc_pool_body(gt_ref, out_ref):
    x = gt_ref[0]
    xf = x.astype(jnp.float32)
    v = xf.reshape(H, BLK, GW // 128, 128).sum(axis=1)
    vm = v.reshape(H, GW).astype(jnp.bfloat16)
    i0 = lax.broadcasted_iota(jnp.int32, (GW, W), 0)
    i1 = lax.broadcasted_iota(jnp.int32, (GW, W), 1)
    p8 = (i0 // BLK == i1).astype(jnp.bfloat16)
    den = lax.dot_general(vm, p8, (((1,), (0,)), ((), ())),
                          preferred_element_type=jnp.float32)
    j0 = lax.broadcasted_iota(jnp.int32, (PR, H), 0)
    j1 = lax.broadcasted_iota(jnp.int32, (PR, H), 1)
    ev = (j1 == 2 * j0).astype(jnp.bfloat16)
    od = (j1 == 2 * j0 + 1).astype(jnp.bfloat16)
    denb = den.astype(jnp.bfloat16)
    de = lax.dot_general(ev, denb, (((1,), (0,)), ((), ())),
                         preferred_element_type=jnp.float32)
    do = lax.dot_general(od, denb, (((1,), (0,)), ((), ())),
                         preferred_element_type=jnp.float32)
    den2 = jnp.concatenate([de, do], axis=1)
    cls = (den2 >= THRESH[0]).astype(jnp.int32)
    for t in THRESH[1:]:
        cls = cls + (den2 >= t).astype(jnp.int32)
    out_ref[0] = cls


@jax.jit
def _tc_pool(gt_lin):
    return pl.pallas_call(
        _tc_pool_body,
        grid=(B - SCK,),
        in_specs=[pl.BlockSpec((1, GH * GW // 128, 128),
                               lambda i: (i + SCK, 0, 0))],
        out_specs=pl.BlockSpec((1, PR, 128), lambda i: (i + SCK, 0, 0)),
        out_shape=jax.ShapeDtypeStruct((B, PR, 128), jnp.int32),
    )(gt_lin)


IB = 8


def _tc_loss_body(logits_ref, cls_sc_ref, cls_tc_ref, out_ref):
    x = logits_ref[...].reshape(IB, C, PR, 128)
    s = jnp.sum(jnp.exp(x), axis=1, keepdims=True)
    lse = jnp.log(s)
    on_sc = pl.program_id(0) * IB < SCK
    cls2 = jnp.where(on_sc, cls_sc_ref[...], cls_tc_ref[...])
    cls = cls2[:, None, :, :]
    tgt = cls - 1
    picked = jnp.zeros_like(lse)
    for cc in range(C):
        picked = picked + jnp.where(tgt == cc, x[:, cc:cc + 1, :, :], 0.0)
    contrib = jnp.sum(jnp.where(cls > 0, lse - picked, 0.0))

    @pl.when(pl.program_id(0) == 0)
    def _():
        out_ref[0, 0] = 0.0

    out_ref[0, 0] += contrib


@jax.jit
def _tc_loss(logits_lin, cls_sc, cls_tc):
    return pl.pallas_call(
        _tc_loss_body,
        grid=(B // IB,),
        in_specs=[
            pl.BlockSpec((IB, C * PR, 128), lambda b: (b, 0, 0)),
            pl.BlockSpec((IB, PR, 128), lambda b: (b, 0, 0)),
            pl.BlockSpec((IB, PR, 128), lambda b: (b, 0, 0)),
        ],
        out_specs=pl.BlockSpec((1, 1), lambda b: (0, 0),
                               memory_space=pltpu.SMEM),
        out_shape=jax.ShapeDtypeStruct((1, 1), jnp.float32),
    )(logits_lin, cls_sc, cls_tc)


def kernel(logit_maps, gt_den_maps):
    cls_sc = _sc_cls(gt_den_maps)
    gt_lin = gt_den_maps.reshape(B, GH * GW // 128, 128)
    cls_tc = _tc_pool(gt_lin)
    logits_lin = logit_maps.reshape(B, C * PR, 128)
    total = _tc_loss(logits_lin, cls_sc, cls_tc)
    loss = total[0, 0] * jnp.float32(1.0 / B)
    return (loss, {"cls_zice": lax.stop_gradient(loss)})


# speedup vs baseline: 1.6792x; 1.6792x over previous
import functools

import jax
import jax.numpy as jnp
from jax import lax
from jax.experimental import pallas as pl
from jax.experimental.pallas import tpu as pltpu
from jax.experimental.pallas import tpu_sc as plsc

B, C, H, W = 32, 7, 64, 64
GH, GW = 512, 512
BLK = 8
HW = H * W
PR = HW // 128
NC, NS, L = 2, 16, 16
SCK = 16
TPI = (NC * NS) // SCK
ROWS_T = GH // TPI
CHUNK = 64
NCHUNK = ROWS_T // CHUNK
BR_PER_CHUNK = CHUNK // BLK
VPR = GW // L
THRESH = (1, 2, 4, 8, 16, 32, 49)


def _sc_cls_body(gt_hbm, cls_hbm, in_buf0, in_buf1, rowsum, out_buf,
                 sem0, sem1):
    wid = lax.axis_index("s") * NC + lax.axis_index("c")
    img = wid // TPI
    half = wid % TPI
    lanes = lax.iota(jnp.int32, L)
    bufs = (in_buf0, in_buf1)
    sems = (sem0, sem1)

    def src_rows(chunk):
        return gt_hbm.at[img, 0, pl.ds(half * ROWS_T + chunk * CHUNK, CHUNK)]

    def compute_chunk(chunk, buf):
        def br_body(br, c2):
            for v0 in range(0, VPR, 8):
                sums = []
                for v in range(v0, v0 + 8):
                    xs = [buf[br * BLK + r, pl.ds(v * L, L)]
                          for r in range(BLK)]
                    while len(xs) > 1:
                        xs = [xs[i] + xs[i + 1] for i in range(0, len(xs), 2)]
                    sums.append(xs[0])
                for dv, s in enumerate(sums):
                    rowsum[pl.ds((v0 + dv) * L, L)] = s
            brl = chunk * BR_PER_CHUNK + br
            orow = brl // 2
            ocol0 = (brl % 2) * W
            for g in range(W // L):
                idx0 = lanes * BLK + g * (L * BLK)
                gs = [plsc.load_gather(rowsum, [idx0 + j]) for j in range(BLK)]
                while len(gs) > 1:
                    gs = [gs[i] + gs[i + 1] for i in range(0, len(gs), 2)]
                acc = gs[0]
                cls = (acc >= THRESH[0]).astype(jnp.int32)
                for t in THRESH[1:]:
                    cls = cls + (acc >= t).astype(jnp.int32)
                out_buf[orow, pl.ds(ocol0 + g * L, L)] = cls
            return c2

        lax.fori_loop(0, BR_PER_CHUNK, br_body, 0)

    pltpu.async_copy(src_rows(0), bufs[0], sems[0])
    pltpu.async_copy(src_rows(1), bufs[1], sems[1])
    for chunk in range(NCHUNK):
        b = chunk % 2
        pltpu.make_async_copy(src_rows(0), bufs[b], sems[b]).wait()
        compute_chunk(chunk, bufs[b])
        if chunk + 2 < NCHUNK:
            pltpu.async_copy(src_rows(chunk + 2), bufs[b], sems[b])
    pltpu.sync_copy(out_buf,
                    cls_hbm.at[img, pl.ds(half * (PR // TPI), PR // TPI)])


@jax.jit
def _sc_cls(gt4):
    mesh = plsc.VectorSubcoreMesh(core_axis_name="c", subcore_axis_name="s",
                                  num_cores=NC, num_subcores=NS)
    return pl.kernel(
        _sc_cls_body,
        out_type=jax.ShapeDtypeStruct((B, PR, 128), jnp.int32),
        mesh=mesh,
        compiler_params=pltpu.CompilerParams(needs_layout_passes=False),
        scratch_types=[
            pltpu.VMEM((CHUNK, GW), jnp.int32),
            pltpu.VMEM((CHUNK, GW), jnp.int32),
            pltpu.VMEM((GW,), jnp.int32),
            pltpu.VMEM((PR // TPI, 128), jnp.int32),
            pltpu.SemaphoreType.DMA,
            pltpu.SemaphoreType.DMA,
        ],
    )(gt4)


def _tc_pool_body(gt_ref, out_ref):
    x = gt_ref[0, 0]
    xf = x.astype(jnp.float32)
    v = xf.reshape(H, BLK, GW).sum(axis=1)
    vm = v.astype(jnp.bfloat16)
    i0 = lax.broadcasted_iota(jnp.int32, (GW, W), 0)
    i1 = lax.broadcasted_iota(jnp.int32, (GW, W), 1)
    p8 = (i0 // BLK == i1).astype(jnp.bfloat16)
    den = lax.dot_general(vm, p8, (((1,), (0,)), ((), ())),
                          preferred_element_type=jnp.float32)
    j0 = lax.broadcasted_iota(jnp.int32, (PR, H), 0)
    j1 = lax.broadcasted_iota(jnp.int32, (PR, H), 1)
    ev = (j1 == 2 * j0).astype(jnp.bfloat16)
    od = (j1 == 2 * j0 + 1).astype(jnp.bfloat16)
    denb = den.astype(jnp.bfloat16)
    de = lax.dot_general(ev, denb, (((1,), (0,)), ((), ())),
                         preferred_element_type=jnp.float32)
    do = lax.dot_general(od, denb, (((1,), (0,)), ((), ())),
                         preferred_element_type=jnp.float32)
    den2 = jnp.concatenate([de, do], axis=1)
    cls = (den2 >= THRESH[0]).astype(jnp.int32)
    for t in THRESH[1:]:
        cls = cls + (den2 >= t).astype(jnp.int32)
    out_ref[0] = cls


@jax.jit
def _tc_pool(gt_lin):
    return pl.pallas_call(
        _tc_pool_body,
        grid=(B - SCK,),
        in_specs=[pl.BlockSpec((1, 1, GH, GW),
                               lambda i: (i + SCK, 0, 0, 0))],
        out_specs=pl.BlockSpec((1, PR, 128), lambda i: (i + SCK, 0, 0)),
        out_shape=jax.ShapeDtypeStruct((B, PR, 128), jnp.int32),
    )(gt_lin)


IB = 8


def _tc_loss_body(logits_ref, cls_sc_ref, cls_tc_ref, out_ref):
    x = logits_ref[...].reshape(IB, C, PR, 128)
    s = jnp.sum(jnp.exp(x), axis=1, keepdims=True)
    lse = jnp.log(s)
    on_sc = pl.program_id(0) * IB < SCK
    cls2 = jnp.where(on_sc, cls_sc_ref[...], cls_tc_ref[...])
    cls = cls2[:, None, :, :]
    tgt = cls - 1
    picked = jnp.zeros_like(lse)
    for cc in range(C):
        picked = picked + jnp.where(tgt == cc, x[:, cc:cc + 1, :, :], 0.0)
    contrib = jnp.sum(jnp.where(cls > 0, lse - picked, 0.0))

    @pl.when(pl.program_id(0) == 0)
    def _():
        out_ref[0, 0] = 0.0

    out_ref[0, 0] += contrib


@jax.jit
def _tc_loss(logits_lin, cls_sc, cls_tc):
    return pl.pallas_call(
        _tc_loss_body,
        grid=(B // IB,),
        in_specs=[
            pl.BlockSpec((IB, C * PR, 128), lambda b: (b, 0, 0)),
            pl.BlockSpec((IB, PR, 128), lambda b: (b, 0, 0)),
            pl.BlockSpec((IB, PR, 128), lambda b: (b, 0, 0)),
        ],
        out_specs=pl.BlockSpec((1, 1), lambda b: (0, 0),
                               memory_space=pltpu.SMEM),
        out_shape=jax.ShapeDtypeStruct((1, 1), jnp.float32),
    )(logits_lin, cls_sc, cls_tc)


def kernel(logit_maps, gt_den_maps):
    cls_sc = _sc_cls(gt_den_maps)
    cls_tc = _tc_pool(gt_den_maps)
    logits_lin = logit_maps.reshape(B, C * PR, 128)
    total = _tc_loss(logits_lin, cls_sc, cls_tc)
    loss = total[0, 0] * jnp.float32(1.0 / B)
    return (loss, {"cls_zice": lax.stop_gradient(loss)})
